# baseline (speedup 1.0000x reference)
import functools

import jax
import jax.numpy as jnp
from jax import lax
from jax.experimental import pallas as pl
from jax.experimental.pallas import tpu as pltpu
from jax.experimental.pallas import tpu_sc as plsc

_BETA = 0.25
_TILE_N = 512


def _argmin_body(flat_ref, emb_ref, idx_ref, loss_ref):
    flat = flat_ref[...]
    emb = emb_ref[...]
    k = emb.shape[0]
    a = jnp.sum(flat * flat, axis=1, keepdims=True)
    b = jnp.sum(emb * emb, axis=1)
    m2 = jax.lax.dot_general(
        flat, -2.0 * emb, (((1,), (1,)), ((), ())),
        preferred_element_type=jnp.float32)
    dist = (a + b) + m2
    mn = jnp.min(dist, axis=1, keepdims=True)
    iota = jax.lax.broadcasted_iota(jnp.int32, (1, k), 1).astype(jnp.float32)
    idx = jnp.min(jnp.where(dist == mn, iota, jnp.float32(k)),
                  axis=1)
    idx_ref[...] = idx.astype(jnp.int32)[None, None, :]
    part = jnp.sum(jnp.broadcast_to(mn, (mn.shape[0], 64)), axis=0,
                   keepdims=True)
    loss_ref[...] = part[None]


def _sc_gather(table, idx):
    n = idx.shape[0]
    d = table.shape[1]
    info = plsc.get_sparse_core_info()
    nw = info.num_cores * info.num_subcores
    b_per_w = n // nw
    mesh = plsc.VectorSubcoreMesh(core_axis_name="c", subcore_axis_name="s")

    @functools.partial(
        pl.kernel, mesh=mesh,
        compiler_params=pltpu.CompilerParams(use_tc_tiling_on_sc=False),
        out_type=jax.ShapeDtypeStruct((n, d), jnp.float32),
        scratch_types=[
            pltpu.VMEM((b_per_w,), jnp.int32),
            pltpu.VMEM((b_per_w, d), jnp.float32),
            pltpu.SemaphoreType.DMA,
        ],
    )
    def gather_kernel(table_hbm, idx_hbm, out_hbm, idx_v, rows_v, sem):
        wid = lax.axis_index("s") * info.num_cores + lax.axis_index("c")
        base = wid * b_per_w
        pltpu.sync_copy(idx_hbm.at[pl.ds(base, b_per_w)], idx_v)
        pltpu.async_copy(table_hbm.at[idx_v], rows_v, sem).wait()
        pltpu.sync_copy(rows_v, out_hbm.at[pl.ds(base, b_per_w)])

    return gather_kernel(table, idx)


def _argmin_call(flat, embedding):
    n, d = flat.shape
    k = embedding.shape[0]
    grid = n // _TILE_N
    return pl.pallas_call(
        _argmin_body,
        grid=(grid,),
        in_specs=[
            pl.BlockSpec((_TILE_N, d), lambda i: (i, 0)),
            pl.BlockSpec((k, d), lambda i: (0, 0)),
        ],
        out_specs=[
            pl.BlockSpec((1, 1, _TILE_N), lambda i: (i, 0, 0)),
            pl.BlockSpec((1, 1, 64), lambda i: (i, 0, 0)),
        ],
        out_shape=[
            jax.ShapeDtypeStruct((grid, 1, _TILE_N), jnp.int32),
            jax.ShapeDtypeStruct((grid, 1, 64), jnp.float32),
        ],
    )(flat, embedding)


_N_CHUNKS = 2


def kernel(latents, embedding):
    batch, d, h, w = latents.shape
    hw = h * w
    flat = jnp.transpose(latents, (0, 2, 3, 1)).reshape(-1, d)
    n = batch * hw
    c = n // _N_CHUNKS
    qs, losses = [], []
    for i in range(_N_CHUNKS):
        idx, loss_parts = _argmin_call(flat[i * c:(i + 1) * c], embedding)
        qs.append(_sc_gather(embedding, idx.reshape(c)))
        losses.append(jnp.sum(loss_parts[:, 0, 0]))
    q = jnp.concatenate(qs, axis=0)
    quantized = q.reshape(batch, hw, d).transpose(0, 2, 1).reshape(
        batch, d, h, w)
    vq_loss = (1.0 + _BETA) * sum(losses) / (n * d)
    return quantized, vq_loss


# speedup vs baseline: 1.2283x; 1.2283x over previous
import functools

import jax
import jax.numpy as jnp
from jax import lax
from jax.experimental import pallas as pl
from jax.experimental.pallas import tpu as pltpu
from jax.experimental.pallas import tpu_sc as plsc

_BETA = 0.25
_TILE_N = 512


def _argmin_body(flat_ref, emb_ref, idx_ref, loss_ref):
    flat = flat_ref[...]
    emb = emb_ref[...]
    k = emb.shape[0]
    a = jnp.sum(flat * flat, axis=1, keepdims=True)
    b = jnp.sum(emb * emb, axis=1)
    m2 = jax.lax.dot_general(
        flat, -2.0 * emb, (((1,), (1,)), ((), ())),
        preferred_element_type=jnp.float32)
    dist = (a + b) + m2
    mn = jnp.min(dist, axis=1, keepdims=True)
    iota = jax.lax.broadcasted_iota(jnp.int32, (1, k), 1).astype(jnp.float32)
    idx = jnp.min(jnp.where(dist == mn, iota, jnp.float32(k)),
                  axis=1)
    idx_ref[...] = idx.astype(jnp.int32)[None, None, :]
    part = jnp.sum(jnp.broadcast_to(mn, (mn.shape[0], 64)), axis=0,
                   keepdims=True)
    loss_ref[...] = part[None]


def _sc_gather(table, idx):
    n = idx.shape[0]
    d = table.shape[1]
    info = plsc.get_sparse_core_info()
    nw = info.num_cores * info.num_subcores
    b_per_w = n // nw
    mesh = plsc.VectorSubcoreMesh(core_axis_name="c", subcore_axis_name="s")

    @functools.partial(
        pl.kernel, mesh=mesh,
        compiler_params=pltpu.CompilerParams(use_tc_tiling_on_sc=False),
        out_type=jax.ShapeDtypeStruct((n, d), jnp.float32),
        scratch_types=[
            pltpu.VMEM((b_per_w,), jnp.int32),
            pltpu.VMEM((b_per_w, d), jnp.float32),
            pltpu.SemaphoreType.DMA,
        ],
    )
    def gather_kernel(table_hbm, idx_hbm, out_hbm, idx_v, rows_v, sem):
        wid = lax.axis_index("s") * info.num_cores + lax.axis_index("c")
        base = wid * b_per_w
        pltpu.sync_copy(idx_hbm.at[pl.ds(base, b_per_w)], idx_v)
        pltpu.async_copy(table_hbm.at[idx_v], rows_v, sem).wait()
        pltpu.sync_copy(rows_v, out_hbm.at[pl.ds(base, b_per_w)])

    return gather_kernel(table, idx)


def _argmin_call(flat, embedding):
    n, d = flat.shape
    k = embedding.shape[0]
    grid = n // _TILE_N
    return pl.pallas_call(
        _argmin_body,
        grid=(grid,),
        in_specs=[
            pl.BlockSpec((_TILE_N, d), lambda i: (i, 0)),
            pl.BlockSpec((k, d), lambda i: (0, 0)),
        ],
        out_specs=[
            pl.BlockSpec((1, 1, _TILE_N), lambda i: (i, 0, 0)),
            pl.BlockSpec((1, 1, 64), lambda i: (i, 0, 0)),
        ],
        out_shape=[
            jax.ShapeDtypeStruct((grid, 1, _TILE_N), jnp.int32),
            jax.ShapeDtypeStruct((grid, 1, 64), jnp.float32),
        ],
    )(flat, embedding)


_N_CHUNKS = 1


def kernel(latents, embedding):
    batch, d, h, w = latents.shape
    hw = h * w
    flat = jnp.transpose(latents, (0, 2, 3, 1)).reshape(-1, d)
    n = batch * hw
    c = n // _N_CHUNKS
    qs, losses = [], []
    for i in range(_N_CHUNKS):
        idx, loss_parts = _argmin_call(flat[i * c:(i + 1) * c], embedding)
        qs.append(_sc_gather(embedding, idx.reshape(c)))
        losses.append(jnp.sum(loss_parts[:, 0, 0]))
    q = jnp.concatenate(qs, axis=0)
    quantized = q.reshape(batch, hw, d).transpose(0, 2, 1).reshape(
        batch, d, h, w)
    vq_loss = (1.0 + _BETA) * sum(losses) / (n * d)
    return quantized, vq_loss
